# Initial kernel scaffold; baseline (speedup 1.0000x reference)
import jax
import jax.numpy as jnp
from jax.experimental import pallas as pl


def kernel(emb, W_ext, b_ext, eps1, eps2, edge_index):
    raise NotImplementedError("write your pallas kernel here")


# speedup vs baseline: 144.1888x; 144.1888x over previous
import functools

import jax
import jax.numpy as jnp
from jax import lax
from jax.experimental import pallas as pl
from jax.experimental.pallas import tpu as pltpu
from jax.experimental.pallas import tpu_sc as plsc

DIM = 4
NPAD = 10240
NW = 32
BLK = 1024
SLICE_C = NPAD // 16
SLICE_W = NPAD // NW
GCHUNK = 64


def _dense_body(embT_ref, WT_ref, b_ref, e1_ref, e2_ref,
                det_ref, pred_ref, z_ref, kcol_ref):
    UT = jnp.dot(WT_ref[...], embT_ref[...],
                 preferred_element_type=jnp.float32) + b_ref[...]
    sig1 = jnp.clip(jax.nn.softplus(UT[0:1, :]), 1e-06, 1e6)
    sig2 = jnp.clip(jax.nn.softplus(UT[1:2, :]), 1e-06, 1e6)
    u = [UT[2 + m:3 + m, :] for m in range(16)]
    ub = [x.astype(jnp.bfloat16).astype(jnp.float32) for x in u]

    P = [[None] * 4 for _ in range(4)]
    for a in range(4):
        for b in range(4):
            if b < a:
                P[a][b] = P[b][a]
                continue
            acc = ub[4 * a + 0] * ub[4 * b + 0]
            for k in range(1, 4):
                acc = acc + ub[4 * a + k] * ub[4 * b + k]
            p = sig1 * acc
            if a == b:
                p = p + sig2
            P[a][b] = p

    def m2(i, j):
        return P[2][i] * P[3][j] - P[2][j] * P[3][i]
    A01, A02, A03 = m2(0, 1), m2(0, 2), m2(0, 3)
    A12, A13, A23 = m2(1, 2), m2(1, 3), m2(2, 3)
    M0 = P[1][1] * A23 - P[1][2] * A13 + P[1][3] * A12
    M1 = P[1][0] * A23 - P[1][2] * A03 + P[1][3] * A02
    M2 = P[1][0] * A13 - P[1][1] * A03 + P[1][3] * A01
    M3 = P[1][0] * A12 - P[1][1] * A02 + P[1][2] * A01
    det = P[0][0] * M0 - P[0][1] * M1 + P[0][2] * M2 - P[0][3] * M3

    trace = P[0][0] + P[1][1] + P[2][2] + P[3][3]
    first = jnp.log(1.0 / (jnp.abs(det) + 1e-06))
    kcol = (first + (-float(DIM)) + trace) / 2.0

    rs1 = jnp.sqrt(sig1)
    rs2 = jnp.sqrt(sig2)
    det_ref[0:1, :] = det
    kcol_ref[0:1, :] = kcol
    for a in range(4):
        acc = u[4 * a + 0] * e1_ref[0:1, :]
        for k in range(1, 4):
            acc = acc + u[4 * a + k] * e1_ref[k:k + 1, :]
        z_ref[a:a + 1, :] = rs1 * acc + rs2 * e2_ref[a:a + 1, :]
    for a in range(4):
        for b in range(4):
            m = 4 * a + b
            pred_ref[m:m + 1, :] = P[a][b]


def _dense_tc(embT, WT, b2, e1T, e2T):
    nblk = NPAD // BLK
    return pl.pallas_call(
        _dense_body,
        grid=(nblk,),
        in_specs=[
            pl.BlockSpec((128, BLK), lambda i: (0, i)),
            pl.BlockSpec((18, 128), lambda i: (0, 0)),
            pl.BlockSpec((18, 1), lambda i: (0, 0)),
            pl.BlockSpec((4, BLK), lambda i: (0, i)),
            pl.BlockSpec((4, BLK), lambda i: (0, i)),
        ],
        out_specs=[
            pl.BlockSpec((1, BLK), lambda i: (0, i)),
            pl.BlockSpec((16, BLK), lambda i: (0, i)),
            pl.BlockSpec((4, BLK), lambda i: (0, i)),
            pl.BlockSpec((1, BLK), lambda i: (0, i)),
        ],
        out_shape=[
            jax.ShapeDtypeStruct((1, NPAD), jnp.float32),
            jax.ShapeDtypeStruct((16, NPAD), jnp.float32),
            jax.ShapeDtypeStruct((4, NPAD), jnp.float32),
            jax.ShapeDtypeStruct((1, NPAD), jnp.float32),
        ],
    )(embT, WT, b2, e1T, e2T)


def _mesh():
    return plsc.VectorSubcoreMesh(core_axis_name="c", subcore_axis_name="s",
                                  num_cores=2, num_subcores=16)


def _lex_better(v_new, a_new, v_old, a_old):
    return (v_new > v_old) | ((v_new == v_old) & (a_new > a_old))


def _sc_partial(det, src, dst):
    E = src.shape[0]
    epw = E // NW
    nvec = epw // 16

    def body(det_hbm, src_hbm, dst_hbm, pv_hbm, pa_hbm,
             det_v, src_v, dst_v, val_v, arg_v,
             tval, targ, aval, aarg, shv, sha):
        cid = lax.axis_index("c")
        sid = lax.axis_index("s")
        wid = cid * 16 + sid
        e0 = wid * epw
        pltpu.sync_copy(det_hbm, det_v)
        pltpu.sync_copy(src_hbm.at[pl.ds(e0, epw)], src_v)
        pltpu.sync_copy(dst_hbm.at[pl.ds(e0, epw)], dst_v)

        neg_inf = jnp.full((16,), -jnp.inf, jnp.float32)
        neg_one = jnp.full((16,), -1, jnp.int32)

        def init(i, _):
            sl = pl.ds(i * 16, 16)
            val_v[sl] = neg_inf
            arg_v[sl] = neg_one
            return 0
        lax.fori_loop(0, NPAD // 16, init, 0)

        lane = lax.iota(jnp.int32, 16)

        def edge(i, _):
            sl = pl.ds(i * 16, 16)
            s_idx = src_v[sl]
            d_idx = dst_v[sl]
            key = plsc.load_gather(det_v, [d_idx])
            eid = e0 + i * 16 + lane

            def wbody(go):
                cur_v = plsc.load_gather(val_v, [s_idx])
                cur_a = plsc.load_gather(arg_v, [s_idx])
                better = _lex_better(key, eid, cur_v, cur_a)
                plsc.store_scatter(val_v, [s_idx], key, mask=better)
                plsc.store_scatter(arg_v, [s_idx], eid, mask=better)
                return jnp.any(better)
            lax.while_loop(lambda go: go, wbody, jnp.bool_(True))
            return 0
        lax.fori_loop(0, nvec, edge, 0)

        pltpu.sync_copy(val_v, shv.at[pl.ds(sid * NPAD, NPAD)])
        pltpu.sync_copy(arg_v, sha.at[pl.ds(sid * NPAD, NPAD)])
        plsc.subcore_barrier()

        off = sid * SLICE_C
        pltpu.sync_copy(shv.at[pl.ds(off, SLICE_C)], aval)
        pltpu.sync_copy(sha.at[pl.ds(off, SLICE_C)], aarg)

        def mt(t, _):
            pltpu.sync_copy(shv.at[pl.ds(t * NPAD + off, SLICE_C)], tval)
            pltpu.sync_copy(sha.at[pl.ds(t * NPAD + off, SLICE_C)], targ)

            def mj(j, _):
                sl = pl.ds(j * 16, 16)
                av, aa = aval[sl], aarg[sl]
                tv, ta = tval[sl], targ[sl]
                better = _lex_better(tv, ta, av, aa)
                aval[sl] = jnp.where(better, tv, av)
                aarg[sl] = jnp.where(better, ta, aa)
                return 0
            lax.fori_loop(0, SLICE_C // 16, mj, 0)
            return 0
        lax.fori_loop(1, 16, mt, 0)

        pltpu.sync_copy(aval, pv_hbm.at[pl.ds(cid * NPAD + off, SLICE_C)])
        pltpu.sync_copy(aarg, pa_hbm.at[pl.ds(cid * NPAD + off, SLICE_C)])

    f = pl.kernel(
        body,
        out_type=(jax.ShapeDtypeStruct((2 * NPAD,), jnp.float32),
                  jax.ShapeDtypeStruct((2 * NPAD,), jnp.int32)),
        mesh=_mesh(),
        compiler_params=pltpu.CompilerParams(needs_layout_passes=False),
        scratch_types=[
            pltpu.VMEM((NPAD,), jnp.float32),
            pltpu.VMEM((epw,), jnp.int32),
            pltpu.VMEM((epw,), jnp.int32),
            pltpu.VMEM((NPAD,), jnp.float32),
            pltpu.VMEM((NPAD,), jnp.int32),
            pltpu.VMEM((SLICE_C,), jnp.float32),
            pltpu.VMEM((SLICE_C,), jnp.int32),
            pltpu.VMEM((SLICE_C,), jnp.float32),
            pltpu.VMEM((SLICE_C,), jnp.int32),
            pltpu.VMEM_SHARED((16 * NPAD,), jnp.float32),
            pltpu.VMEM_SHARED((16 * NPAD,), jnp.int32),
        ],
    )
    return f(det, src, dst)


def _merge_and_arg(pv_hbm, pa_hbm, v0, v1, a0, a1, argm, off):
    pltpu.sync_copy(pv_hbm.at[pl.ds(off, SLICE_W)], v0)
    pltpu.sync_copy(pv_hbm.at[pl.ds(NPAD + off, SLICE_W)], v1)
    pltpu.sync_copy(pa_hbm.at[pl.ds(off, SLICE_W)], a0)
    pltpu.sync_copy(pa_hbm.at[pl.ds(NPAD + off, SLICE_W)], a1)

    def mj(j, _):
        sl = pl.ds(j * 16, 16)
        b = _lex_better(v1[sl], a1[sl], v0[sl], a0[sl])
        am = jnp.where(b, a1[sl], a0[sl])
        argm[sl] = jnp.maximum(am, 0)
        return 0
    lax.fori_loop(0, SLICE_W // 16, mj, 0)


def _gather_rows_1d(tab_hbm, idx_ref, out_ref, sem):
    descs = []
    for ch in range(SLICE_W // GCHUNK):
        sl = pl.ds(ch * GCHUNK, GCHUNK)
        descs.append(pltpu.async_copy(tab_hbm.at[idx_ref.at[sl]],
                                      out_ref.at[sl], sem))
    for d in descs:
        d.wait()


def _sc_merge1(pv, pa, dst, det):
    def body(pv_hbm, pa_hbm, dst_hbm, det_hbm, sel_hbm, det2_hbm,
             v0, v1, a0, a1, argm, sel_v, dtab, det2_v, sem):
        cid = lax.axis_index("c")
        sid = lax.axis_index("s")
        wid = cid * 16 + sid
        off = wid * SLICE_W
        _merge_and_arg(pv_hbm, pa_hbm, v0, v1, a0, a1, argm, off)
        _gather_rows_1d(dst_hbm, argm, sel_v, sem)
        pltpu.sync_copy(det_hbm, dtab)

        def gj(j, _):
            sl = pl.ds(j * 16, 16)
            det2_v[sl] = plsc.load_gather(dtab, [sel_v[sl]])
            return 0
        lax.fori_loop(0, SLICE_W // 16, gj, 0)
        pltpu.sync_copy(sel_v, sel_hbm.at[pl.ds(off, SLICE_W)])
        pltpu.sync_copy(det2_v, det2_hbm.at[pl.ds(off, SLICE_W)])

    f = pl.kernel(
        body,
        out_type=(jax.ShapeDtypeStruct((NPAD,), jnp.int32),
                  jax.ShapeDtypeStruct((NPAD,), jnp.float32)),
        mesh=_mesh(),
        compiler_params=pltpu.CompilerParams(needs_layout_passes=False),
        scratch_types=[
            pltpu.VMEM((SLICE_W,), jnp.float32),
            pltpu.VMEM((SLICE_W,), jnp.float32),
            pltpu.VMEM((SLICE_W,), jnp.int32),
            pltpu.VMEM((SLICE_W,), jnp.int32),
            pltpu.VMEM((SLICE_W,), jnp.int32),
            pltpu.VMEM((SLICE_W,), jnp.int32),
            pltpu.VMEM((NPAD,), jnp.float32),
            pltpu.VMEM((SLICE_W,), jnp.float32),
            pltpu.SemaphoreType.DMA,
        ],
    )
    return f(pv, pa, dst, det)


def _sc_final(pv, pa, dst, sel1, tbl):
    def body(pv_hbm, pa_hbm, dst_hbm, sel1_hbm, tbl_hbm,
             rows_hbm,
             v0, v1, a0, a1, argm, sel2_v, s1tab, sf_v, rows_v, sem):
        cid = lax.axis_index("c")
        sid = lax.axis_index("s")
        wid = cid * 16 + sid
        off = wid * SLICE_W
        _merge_and_arg(pv_hbm, pa_hbm, v0, v1, a0, a1, argm, off)
        _gather_rows_1d(dst_hbm, argm, sel2_v, sem)
        pltpu.sync_copy(sel1_hbm, s1tab)

        def gj(j, _):
            sl = pl.ds(j * 16, 16)
            sf_v[sl] = plsc.load_gather(s1tab, [sel2_v[sl]])
            return 0
        lax.fori_loop(0, SLICE_W // 16, gj, 0)

        descs = []
        for ch in range(SLICE_W // GCHUNK):
            sl = pl.ds(ch * GCHUNK, GCHUNK)
            descs.append(pltpu.async_copy(tbl_hbm.at[sf_v.at[sl]],
                                          rows_v.at[sl, :], sem))
        for d in descs:
            d.wait()
        pltpu.sync_copy(rows_v, rows_hbm.at[pl.ds(off, SLICE_W), :])

    f = pl.kernel(
        body,
        out_type=jax.ShapeDtypeStruct((NPAD, 128), jnp.float32),
        mesh=_mesh(),
        compiler_params=pltpu.CompilerParams(needs_layout_passes=False),
        scratch_types=[
            pltpu.VMEM((SLICE_W,), jnp.float32),
            pltpu.VMEM((SLICE_W,), jnp.float32),
            pltpu.VMEM((SLICE_W,), jnp.int32),
            pltpu.VMEM((SLICE_W,), jnp.int32),
            pltpu.VMEM((SLICE_W,), jnp.int32),
            pltpu.VMEM((SLICE_W,), jnp.int32),
            pltpu.VMEM((NPAD,), jnp.int32),
            pltpu.VMEM((SLICE_W,), jnp.int32),
            pltpu.VMEM((SLICE_W, 128), jnp.float32),
            pltpu.SemaphoreType.DMA,
        ],
    )
    return f(pv, pa, dst, sel1, tbl)


def kernel(emb, W_ext, b_ext, eps1, eps2, edge_index):
    N = emb.shape[0]
    embT = jnp.pad(emb, ((0, NPAD - N), (0, 0))).T
    WT = W_ext.T
    b2 = b_ext[:, None]
    e1T = jnp.pad(eps1[:, :, 0], ((0, NPAD - N), (0, 0))).T
    e2T = jnp.pad(eps2[:, :, 0], ((0, NPAD - N), (0, 0))).T

    detT, predT, zT, kcolT = _dense_tc(embT, WT, b2, e1T, e2T)
    det1 = detT[0]
    z = zT.T[:N]
    tbl = jnp.pad(jnp.concatenate([predT, kcolT], axis=0).T,
                  ((0, 0), (0, 128 - 17)))

    src_n = edge_index[0]
    dst_n = edge_index[1]
    pv, pa = _sc_partial(det1, src_n, dst_n)
    sel1, det2 = _sc_merge1(pv, pa, dst_n, det1)
    pv2, pa2 = _sc_partial(det2, src_n, dst_n)
    rows = _sc_final(pv2, pa2, dst_n, sel1, tbl)

    g = rows[:N]
    return jnp.concatenate([g[:, :16], z, g[:, 16:17]], axis=1)


# speedup vs baseline: 228.8182x; 1.5869x over previous
import functools

import jax
import jax.numpy as jnp
from jax import lax
from jax.experimental import pallas as pl
from jax.experimental.pallas import tpu as pltpu
from jax.experimental.pallas import tpu_sc as plsc

DIM = 4
NPAD = 10240
NW = 32
BLK = 1024
SLICE_C = NPAD // 16
SLICE_W = NPAD // NW
GCHUNK = 64


def _dense_body(embT_ref, WT_ref, b_ref, e1_ref, e2_ref,
                det_ref, pred_ref, z_ref, kcol_ref):
    UT = lax.dot_general(WT_ref[...], embT_ref[...], (((1,), (1,)), ((), ())),
                         preferred_element_type=jnp.float32) + b_ref[...]
    sig1 = jnp.clip(jax.nn.softplus(UT[0:1, :]), 1e-06, 1e6)
    sig2 = jnp.clip(jax.nn.softplus(UT[1:2, :]), 1e-06, 1e6)
    u = [UT[2 + m:3 + m, :] for m in range(16)]
    ub = [x.astype(jnp.bfloat16).astype(jnp.float32) for x in u]

    P = [[None] * 4 for _ in range(4)]
    for a in range(4):
        for b in range(4):
            if b < a:
                P[a][b] = P[b][a]
                continue
            acc = ub[4 * a + 0] * ub[4 * b + 0]
            for k in range(1, 4):
                acc = acc + ub[4 * a + k] * ub[4 * b + k]
            p = sig1 * acc
            if a == b:
                p = p + sig2
            P[a][b] = p

    def m2(i, j):
        return P[2][i] * P[3][j] - P[2][j] * P[3][i]
    A01, A02, A03 = m2(0, 1), m2(0, 2), m2(0, 3)
    A12, A13, A23 = m2(1, 2), m2(1, 3), m2(2, 3)
    M0 = P[1][1] * A23 - P[1][2] * A13 + P[1][3] * A12
    M1 = P[1][0] * A23 - P[1][2] * A03 + P[1][3] * A02
    M2 = P[1][0] * A13 - P[1][1] * A03 + P[1][3] * A01
    M3 = P[1][0] * A12 - P[1][1] * A02 + P[1][2] * A01
    det = P[0][0] * M0 - P[0][1] * M1 + P[0][2] * M2 - P[0][3] * M3

    trace = P[0][0] + P[1][1] + P[2][2] + P[3][3]
    first = jnp.log(1.0 / (jnp.abs(det) + 1e-06))
    kcol = (first + (-float(DIM)) + trace) / 2.0

    rs1 = jnp.sqrt(sig1)
    rs2 = jnp.sqrt(sig2)
    det_ref[0:1, :] = det
    kcol_ref[0:1, :] = kcol
    for a in range(4):
        acc = u[4 * a + 0] * e1_ref[0:1, :]
        for k in range(1, 4):
            acc = acc + u[4 * a + k] * e1_ref[k:k + 1, :]
        z_ref[a:a + 1, :] = rs1 * acc + rs2 * e2_ref[a:a + 1, :]
    for a in range(4):
        for b in range(4):
            m = 4 * a + b
            pred_ref[m:m + 1, :] = P[a][b]


def _dense_tc(embT, WT, b2, e1T, e2T):
    nblk = NPAD // BLK
    return pl.pallas_call(
        _dense_body,
        grid=(nblk,),
        in_specs=[
            pl.BlockSpec((BLK, 128), lambda i: (i, 0)),
            pl.BlockSpec((18, 128), lambda i: (0, 0)),
            pl.BlockSpec((18, 1), lambda i: (0, 0)),
            pl.BlockSpec((4, BLK), lambda i: (0, i)),
            pl.BlockSpec((4, BLK), lambda i: (0, i)),
        ],
        out_specs=[
            pl.BlockSpec((1, BLK), lambda i: (0, i)),
            pl.BlockSpec((16, BLK), lambda i: (0, i)),
            pl.BlockSpec((4, BLK), lambda i: (0, i)),
            pl.BlockSpec((1, BLK), lambda i: (0, i)),
        ],
        out_shape=[
            jax.ShapeDtypeStruct((1, NPAD), jnp.float32),
            jax.ShapeDtypeStruct((16, NPAD), jnp.float32),
            jax.ShapeDtypeStruct((4, NPAD), jnp.float32),
            jax.ShapeDtypeStruct((1, NPAD), jnp.float32),
        ],
    )(embT, WT, b2, e1T, e2T)


def _mesh():
    return plsc.VectorSubcoreMesh(core_axis_name="c", subcore_axis_name="s",
                                  num_cores=2, num_subcores=16)


def _lex_better(v_new, a_new, v_old, a_old):
    return (v_new > v_old) | ((v_new == v_old) & (a_new > a_old))


def _sc_partial(det, src, dst):
    E = src.shape[0]
    epw = E // NW
    nvec = epw // 16

    def body(det_hbm, src_hbm, dst_hbm, pv_hbm, pa_hbm,
             det_v, src_v, dst_v, val_v, arg_v,
             tval, targ, aval, aarg, shv, sha, sem):
        cid = lax.axis_index("c")
        sid = lax.axis_index("s")
        wid = cid * 16 + sid
        e0 = wid * epw
        stage = [pltpu.async_copy(det_hbm, det_v, sem),
                 pltpu.async_copy(src_hbm.at[pl.ds(e0, epw)], src_v, sem),
                 pltpu.async_copy(dst_hbm.at[pl.ds(e0, epw)], dst_v, sem)]
        for d in stage:
            d.wait()

        neg_inf = jnp.full((16,), -jnp.inf, jnp.float32)
        neg_one = jnp.full((16,), -1, jnp.int32)

        def init(i, _):
            sl = pl.ds(i * 16, 16)
            val_v[sl] = neg_inf
            arg_v[sl] = neg_one
            return 0
        lax.fori_loop(0, NPAD // 16, init, 0)

        lane = lax.iota(jnp.int32, 16)

        def edge2(i, unres):
            sl = pl.ds(i * 16, 16)
            s_idx = src_v[sl]
            d_idx = dst_v[sl]
            key = plsc.load_gather(det_v, [d_idx])
            eid = e0 + i * 16 + lane
            for _ in range(2):
                cur_v = plsc.load_gather(val_v, [s_idx])
                cur_a = plsc.load_gather(arg_v, [s_idx])
                better = _lex_better(key, eid, cur_v, cur_a)
                plsc.store_scatter(val_v, [s_idx], key, mask=better)
                plsc.store_scatter(arg_v, [s_idx], eid, mask=better)
            cur_v = plsc.load_gather(val_v, [s_idx])
            cur_a = plsc.load_gather(arg_v, [s_idx])
            better = _lex_better(key, eid, cur_v, cur_a)
            return unres | jnp.where(better, 1, 0)
        unres = lax.fori_loop(0, nvec, edge2, jnp.zeros((16,), jnp.int32))

        @pl.when(jnp.any(unres != 0))
        def _fixpoint():
            def edge_fix(i, _):
                sl = pl.ds(i * 16, 16)
                s_idx = src_v[sl]
                d_idx = dst_v[sl]
                key = plsc.load_gather(det_v, [d_idx])
                eid = e0 + i * 16 + lane

                def wbody(go):
                    cur_v = plsc.load_gather(val_v, [s_idx])
                    cur_a = plsc.load_gather(arg_v, [s_idx])
                    better = _lex_better(key, eid, cur_v, cur_a)
                    plsc.store_scatter(val_v, [s_idx], key, mask=better)
                    plsc.store_scatter(arg_v, [s_idx], eid, mask=better)
                    return jnp.any(better)
                lax.while_loop(lambda go: go, wbody, jnp.bool_(True))
                return 0
            lax.fori_loop(0, nvec, edge_fix, 0)

        pltpu.sync_copy(val_v, shv.at[pl.ds(sid * NPAD, NPAD)])
        pltpu.sync_copy(arg_v, sha.at[pl.ds(sid * NPAD, NPAD)])
        plsc.subcore_barrier()

        off = sid * SLICE_C
        pltpu.sync_copy(shv.at[pl.ds(off, SLICE_C)], aval)
        pltpu.sync_copy(sha.at[pl.ds(off, SLICE_C)], aarg)

        def mt(t, _):
            pltpu.sync_copy(shv.at[pl.ds(t * NPAD + off, SLICE_C)], tval)
            pltpu.sync_copy(sha.at[pl.ds(t * NPAD + off, SLICE_C)], targ)

            def mj(j, _):
                sl = pl.ds(j * 16, 16)
                av, aa = aval[sl], aarg[sl]
                tv, ta = tval[sl], targ[sl]
                better = _lex_better(tv, ta, av, aa)
                aval[sl] = jnp.where(better, tv, av)
                aarg[sl] = jnp.where(better, ta, aa)
                return 0
            lax.fori_loop(0, SLICE_C // 16, mj, 0)
            return 0
        lax.fori_loop(1, 16, mt, 0)

        pltpu.sync_copy(aval, pv_hbm.at[pl.ds(cid * NPAD + off, SLICE_C)])
        pltpu.sync_copy(aarg, pa_hbm.at[pl.ds(cid * NPAD + off, SLICE_C)])

    f = pl.kernel(
        body,
        out_type=(jax.ShapeDtypeStruct((2 * NPAD,), jnp.float32),
                  jax.ShapeDtypeStruct((2 * NPAD,), jnp.int32)),
        mesh=_mesh(),
        compiler_params=pltpu.CompilerParams(needs_layout_passes=False),
        scratch_types=[
            pltpu.VMEM((NPAD,), jnp.float32),
            pltpu.VMEM((epw,), jnp.int32),
            pltpu.VMEM((epw,), jnp.int32),
            pltpu.VMEM((NPAD,), jnp.float32),
            pltpu.VMEM((NPAD,), jnp.int32),
            pltpu.VMEM((SLICE_C,), jnp.float32),
            pltpu.VMEM((SLICE_C,), jnp.int32),
            pltpu.VMEM((SLICE_C,), jnp.float32),
            pltpu.VMEM((SLICE_C,), jnp.int32),
            pltpu.VMEM_SHARED((16 * NPAD,), jnp.float32),
            pltpu.VMEM_SHARED((16 * NPAD,), jnp.int32),
            pltpu.SemaphoreType.DMA,
        ],
    )
    return f(det, src, dst)


def _merge_and_arg(pv_hbm, pa_hbm, v0, v1, a0, a1, argm, off, sem):
    stage = [pltpu.async_copy(pv_hbm.at[pl.ds(off, SLICE_W)], v0, sem),
             pltpu.async_copy(pv_hbm.at[pl.ds(NPAD + off, SLICE_W)], v1, sem),
             pltpu.async_copy(pa_hbm.at[pl.ds(off, SLICE_W)], a0, sem),
             pltpu.async_copy(pa_hbm.at[pl.ds(NPAD + off, SLICE_W)], a1, sem)]
    for d in stage:
        d.wait()

    def mj(j, _):
        sl = pl.ds(j * 16, 16)
        b = _lex_better(v1[sl], a1[sl], v0[sl], a0[sl])
        am = jnp.where(b, a1[sl], a0[sl])
        argm[sl] = jnp.maximum(am, 0)
        return 0
    lax.fori_loop(0, SLICE_W // 16, mj, 0)


def _gather_rows_1d(tab_hbm, idx_ref, out_ref, sem):
    descs = []
    for ch in range(SLICE_W // GCHUNK):
        sl = pl.ds(ch * GCHUNK, GCHUNK)
        descs.append(pltpu.async_copy(tab_hbm.at[idx_ref.at[sl]],
                                      out_ref.at[sl], sem))
    for d in descs:
        d.wait()


def _sc_merge1(pv, pa, dst, det):
    def body(pv_hbm, pa_hbm, dst_hbm, det_hbm, sel_hbm, det2_hbm,
             v0, v1, a0, a1, argm, sel_v, dtab, det2_v, sem):
        cid = lax.axis_index("c")
        sid = lax.axis_index("s")
        wid = cid * 16 + sid
        off = wid * SLICE_W
        _merge_and_arg(pv_hbm, pa_hbm, v0, v1, a0, a1, argm, off, sem)
        _gather_rows_1d(dst_hbm, argm, sel_v, sem)
        pltpu.sync_copy(det_hbm, dtab)

        def gj(j, _):
            sl = pl.ds(j * 16, 16)
            det2_v[sl] = plsc.load_gather(dtab, [sel_v[sl]])
            return 0
        lax.fori_loop(0, SLICE_W // 16, gj, 0)
        pltpu.sync_copy(sel_v, sel_hbm.at[pl.ds(off, SLICE_W)])
        pltpu.sync_copy(det2_v, det2_hbm.at[pl.ds(off, SLICE_W)])

    f = pl.kernel(
        body,
        out_type=(jax.ShapeDtypeStruct((NPAD,), jnp.int32),
                  jax.ShapeDtypeStruct((NPAD,), jnp.float32)),
        mesh=_mesh(),
        compiler_params=pltpu.CompilerParams(needs_layout_passes=False),
        scratch_types=[
            pltpu.VMEM((SLICE_W,), jnp.float32),
            pltpu.VMEM((SLICE_W,), jnp.float32),
            pltpu.VMEM((SLICE_W,), jnp.int32),
            pltpu.VMEM((SLICE_W,), jnp.int32),
            pltpu.VMEM((SLICE_W,), jnp.int32),
            pltpu.VMEM((SLICE_W,), jnp.int32),
            pltpu.VMEM((NPAD,), jnp.float32),
            pltpu.VMEM((SLICE_W,), jnp.float32),
            pltpu.SemaphoreType.DMA,
        ],
    )
    return f(pv, pa, dst, det)


def _sc_final(pv, pa, dst, sel1, tbl, kcol):
    def body(pv_hbm, pa_hbm, dst_hbm, sel1_hbm, tbl_hbm, kcol_hbm,
             rows_hbm, kl_hbm,
             v0, v1, a0, a1, argm, sel2_v, s1tab, ktab, sf_v, kl_v,
             rows_v, sem):
        cid = lax.axis_index("c")
        sid = lax.axis_index("s")
        wid = cid * 16 + sid
        off = wid * SLICE_W
        _merge_and_arg(pv_hbm, pa_hbm, v0, v1, a0, a1, argm, off, sem)
        _gather_rows_1d(dst_hbm, argm, sel2_v, sem)
        pltpu.sync_copy(sel1_hbm, s1tab)
        pltpu.sync_copy(kcol_hbm, ktab)

        def gj(j, _):
            sl = pl.ds(j * 16, 16)
            sf = plsc.load_gather(s1tab, [sel2_v[sl]])
            sf_v[sl] = sf
            kl_v[sl] = plsc.load_gather(ktab, [sf])
            return 0
        lax.fori_loop(0, SLICE_W // 16, gj, 0)

        descs = []
        for ch in range(SLICE_W // GCHUNK):
            sl = pl.ds(ch * GCHUNK, GCHUNK)
            descs.append(pltpu.async_copy(tbl_hbm.at[sf_v.at[sl]],
                                          rows_v.at[sl, :], sem))
        for d in descs:
            d.wait()
        pltpu.sync_copy(rows_v, rows_hbm.at[pl.ds(off, SLICE_W), :])
        pltpu.sync_copy(kl_v, kl_hbm.at[pl.ds(off, SLICE_W)])

    f = pl.kernel(
        body,
        out_type=(jax.ShapeDtypeStruct((NPAD, 16), jnp.float32),
                  jax.ShapeDtypeStruct((NPAD,), jnp.float32)),
        mesh=_mesh(),
        compiler_params=pltpu.CompilerParams(needs_layout_passes=False,
                                             use_tc_tiling_on_sc=False),
        scratch_types=[
            pltpu.VMEM((SLICE_W,), jnp.float32),
            pltpu.VMEM((SLICE_W,), jnp.float32),
            pltpu.VMEM((SLICE_W,), jnp.int32),
            pltpu.VMEM((SLICE_W,), jnp.int32),
            pltpu.VMEM((SLICE_W,), jnp.int32),
            pltpu.VMEM((SLICE_W,), jnp.int32),
            pltpu.VMEM((NPAD,), jnp.int32),
            pltpu.VMEM((NPAD,), jnp.float32),
            pltpu.VMEM((SLICE_W,), jnp.int32),
            pltpu.VMEM((SLICE_W,), jnp.float32),
            pltpu.VMEM((SLICE_W, 16), jnp.float32),
            pltpu.SemaphoreType.DMA,
        ],
    )
    return f(pv, pa, dst, sel1, tbl, kcol)


def kernel(emb, W_ext, b_ext, eps1, eps2, edge_index):
    N = emb.shape[0]
    embT = jnp.pad(emb, ((0, NPAD - N), (0, 0)))
    WT = W_ext.T
    b2 = b_ext[:, None]
    e1T = jnp.pad(eps1[:, :, 0], ((0, NPAD - N), (0, 0))).T
    e2T = jnp.pad(eps2[:, :, 0], ((0, NPAD - N), (0, 0))).T

    detT, predT, zT, kcolT = _dense_tc(embT, WT, b2, e1T, e2T)
    det1 = detT[0]
    z = zT.T[:N]
    tbl = predT.T
    kcol = kcolT[0]

    src_n = edge_index[0]
    dst_n = edge_index[1]
    pv, pa = _sc_partial(det1, src_n, dst_n)
    sel1, det2 = _sc_merge1(pv, pa, dst_n, det1)
    pv2, pa2 = _sc_partial(det2, src_n, dst_n)
    rows, klsel = _sc_final(pv2, pa2, dst_n, sel1, tbl, kcol)

    return jnp.concatenate([rows[:N], z, klsel[:N, None]], axis=1)


# speedup vs baseline: 236.0049x; 1.0314x over previous
import functools

import jax
import jax.numpy as jnp
from jax import lax
from jax.experimental import pallas as pl
from jax.experimental.pallas import tpu as pltpu
from jax.experimental.pallas import tpu_sc as plsc

DIM = 4
NPAD = 10240
NW = 32
BLK = 1024
SLICE_C = NPAD // 16
SLICE_W = NPAD // NW
GCHUNK = 64


def _dense_body(embT_ref, WT_ref, b_ref, e1_ref, e2_ref,
                det_ref, pred_ref, z_ref, kcol_ref):
    UT = lax.dot_general(WT_ref[...], embT_ref[...], (((1,), (1,)), ((), ())),
                         preferred_element_type=jnp.float32) + b_ref[...]
    sig1 = jnp.clip(jax.nn.softplus(UT[0:1, :]), 1e-06, 1e6)
    sig2 = jnp.clip(jax.nn.softplus(UT[1:2, :]), 1e-06, 1e6)
    u = [UT[2 + m:3 + m, :] for m in range(16)]
    ub = [x.astype(jnp.bfloat16).astype(jnp.float32) for x in u]

    P = [[None] * 4 for _ in range(4)]
    for a in range(4):
        for b in range(4):
            if b < a:
                P[a][b] = P[b][a]
                continue
            acc = ub[4 * a + 0] * ub[4 * b + 0]
            for k in range(1, 4):
                acc = acc + ub[4 * a + k] * ub[4 * b + k]
            p = sig1 * acc
            if a == b:
                p = p + sig2
            P[a][b] = p

    def m2(i, j):
        return P[2][i] * P[3][j] - P[2][j] * P[3][i]
    A01, A02, A03 = m2(0, 1), m2(0, 2), m2(0, 3)
    A12, A13, A23 = m2(1, 2), m2(1, 3), m2(2, 3)
    M0 = P[1][1] * A23 - P[1][2] * A13 + P[1][3] * A12
    M1 = P[1][0] * A23 - P[1][2] * A03 + P[1][3] * A02
    M2 = P[1][0] * A13 - P[1][1] * A03 + P[1][3] * A01
    M3 = P[1][0] * A12 - P[1][1] * A02 + P[1][2] * A01
    det = P[0][0] * M0 - P[0][1] * M1 + P[0][2] * M2 - P[0][3] * M3

    trace = P[0][0] + P[1][1] + P[2][2] + P[3][3]
    first = jnp.log(1.0 / (jnp.abs(det) + 1e-06))
    kcol = (first + (-float(DIM)) + trace) / 2.0

    rs1 = jnp.sqrt(sig1)
    rs2 = jnp.sqrt(sig2)
    det_ref[0:1, :] = det
    kcol_ref[0:1, :] = kcol
    for a in range(4):
        acc = u[4 * a + 0] * e1_ref[0:1, :]
        for k in range(1, 4):
            acc = acc + u[4 * a + k] * e1_ref[k:k + 1, :]
        z_ref[a:a + 1, :] = rs1 * acc + rs2 * e2_ref[a:a + 1, :]
    for a in range(4):
        for b in range(4):
            m = 4 * a + b
            pred_ref[m:m + 1, :] = P[a][b]


def _dense_tc(embT, WT, b2, e1T, e2T):
    nblk = NPAD // BLK
    return pl.pallas_call(
        _dense_body,
        grid=(nblk,),
        in_specs=[
            pl.BlockSpec((BLK, 128), lambda i: (i, 0)),
            pl.BlockSpec((18, 128), lambda i: (0, 0)),
            pl.BlockSpec((18, 1), lambda i: (0, 0)),
            pl.BlockSpec((4, BLK), lambda i: (0, i)),
            pl.BlockSpec((4, BLK), lambda i: (0, i)),
        ],
        out_specs=[
            pl.BlockSpec((1, BLK), lambda i: (0, i)),
            pl.BlockSpec((16, BLK), lambda i: (0, i)),
            pl.BlockSpec((4, BLK), lambda i: (0, i)),
            pl.BlockSpec((1, BLK), lambda i: (0, i)),
        ],
        out_shape=[
            jax.ShapeDtypeStruct((1, NPAD), jnp.float32),
            jax.ShapeDtypeStruct((16, NPAD), jnp.float32),
            jax.ShapeDtypeStruct((4, NPAD), jnp.float32),
            jax.ShapeDtypeStruct((1, NPAD), jnp.float32),
        ],
    )(embT, WT, b2, e1T, e2T)


def _mesh():
    return plsc.VectorSubcoreMesh(core_axis_name="c", subcore_axis_name="s",
                                  num_cores=2, num_subcores=16)


def _lex_better(v_new, a_new, v_old, a_old):
    return (v_new > v_old) | ((v_new == v_old) & (a_new > a_old))


def _sc_partial(det, src, dst):
    E = src.shape[0]
    epw = E // NW
    nvec = epw // 16

    def body(det_hbm, src_hbm, dst_hbm, pv_hbm, pa_hbm,
             det_v, src_v, dst_v, val_v, arg_v,
             tval, targ, aval, aarg, tv2, ta2, shv, sha, sem):
        cid = lax.axis_index("c")
        sid = lax.axis_index("s")
        wid = cid * 16 + sid
        e0 = wid * epw
        stage = [pltpu.async_copy(det_hbm, det_v, sem),
                 pltpu.async_copy(src_hbm.at[pl.ds(e0, epw)], src_v, sem),
                 pltpu.async_copy(dst_hbm.at[pl.ds(e0, epw)], dst_v, sem)]
        for d in stage:
            d.wait()

        neg_inf = jnp.full((16,), -jnp.inf, jnp.float32)
        neg_one = jnp.full((16,), -1, jnp.int32)

        def init(i, _):
            sl = pl.ds(i * 16, 16)
            val_v[sl] = neg_inf
            arg_v[sl] = neg_one
            return 0
        lax.fori_loop(0, NPAD // 16, init, 0)

        lane = lax.iota(jnp.int32, 16)

        def edge2(i, unres):
            sl = pl.ds(i * 16, 16)
            s_idx = src_v[sl]
            d_idx = dst_v[sl]
            key = plsc.load_gather(det_v, [d_idx])
            eid = e0 + i * 16 + lane
            for _ in range(2):
                cur_v = plsc.load_gather(val_v, [s_idx])
                cur_a = plsc.load_gather(arg_v, [s_idx])
                better = _lex_better(key, eid, cur_v, cur_a)
                plsc.store_scatter(val_v, [s_idx], key, mask=better)
                plsc.store_scatter(arg_v, [s_idx], eid, mask=better)
            cur_v = plsc.load_gather(val_v, [s_idx])
            cur_a = plsc.load_gather(arg_v, [s_idx])
            better = _lex_better(key, eid, cur_v, cur_a)
            return unres | jnp.where(better, 1, 0)
        unres = lax.fori_loop(0, nvec, edge2, jnp.zeros((16,), jnp.int32))

        @pl.when(jnp.any(unres != 0))
        def _fixpoint():
            def edge_fix(i, _):
                sl = pl.ds(i * 16, 16)
                s_idx = src_v[sl]
                d_idx = dst_v[sl]
                key = plsc.load_gather(det_v, [d_idx])
                eid = e0 + i * 16 + lane

                def wbody(go):
                    cur_v = plsc.load_gather(val_v, [s_idx])
                    cur_a = plsc.load_gather(arg_v, [s_idx])
                    better = _lex_better(key, eid, cur_v, cur_a)
                    plsc.store_scatter(val_v, [s_idx], key, mask=better)
                    plsc.store_scatter(arg_v, [s_idx], eid, mask=better)
                    return jnp.any(better)
                lax.while_loop(lambda go: go, wbody, jnp.bool_(True))
                return 0
            lax.fori_loop(0, nvec, edge_fix, 0)

        pltpu.sync_copy(val_v, shv.at[pl.ds(sid * NPAD, NPAD)])
        pltpu.sync_copy(arg_v, sha.at[pl.ds(sid * NPAD, NPAD)])
        plsc.subcore_barrier()

        off = sid * SLICE_C
        pltpu.sync_copy(shv.at[pl.ds(off, SLICE_C)], aval)
        pltpu.sync_copy(sha.at[pl.ds(off, SLICE_C)], aarg)

        bufs = ((tval, targ), (tv2, ta2))

        def fire(t, b):
            tv, ta = bufs[b]
            return (pltpu.async_copy(shv.at[pl.ds(t * NPAD + off, SLICE_C)],
                                     tv, sem),
                    pltpu.async_copy(sha.at[pl.ds(t * NPAD + off, SLICE_C)],
                                     ta, sem))
        pend = fire(1, 0)
        for t in range(1, 16):
            b = (t - 1) & 1
            for d in pend:
                d.wait()
            if t < 15:
                nxt = fire(t + 1, 1 - b)
            tv, ta = bufs[b]

            def mj(j, _, tv=tv, ta=ta):
                sl = pl.ds(j * 16, 16)
                av, aa = aval[sl], aarg[sl]
                better = _lex_better(tv[sl], ta[sl], av, aa)
                aval[sl] = jnp.where(better, tv[sl], av)
                aarg[sl] = jnp.where(better, ta[sl], aa)
                return 0
            lax.fori_loop(0, SLICE_C // 16, mj, 0)
            if t < 15:
                pend = nxt

        pltpu.sync_copy(aval, pv_hbm.at[pl.ds(cid * NPAD + off, SLICE_C)])
        pltpu.sync_copy(aarg, pa_hbm.at[pl.ds(cid * NPAD + off, SLICE_C)])

    f = pl.kernel(
        body,
        out_type=(jax.ShapeDtypeStruct((2 * NPAD,), jnp.float32),
                  jax.ShapeDtypeStruct((2 * NPAD,), jnp.int32)),
        mesh=_mesh(),
        compiler_params=pltpu.CompilerParams(needs_layout_passes=False),
        scratch_types=[
            pltpu.VMEM((NPAD,), jnp.float32),
            pltpu.VMEM((epw,), jnp.int32),
            pltpu.VMEM((epw,), jnp.int32),
            pltpu.VMEM((NPAD,), jnp.float32),
            pltpu.VMEM((NPAD,), jnp.int32),
            pltpu.VMEM((SLICE_C,), jnp.float32),
            pltpu.VMEM((SLICE_C,), jnp.int32),
            pltpu.VMEM((SLICE_C,), jnp.float32),
            pltpu.VMEM((SLICE_C,), jnp.int32),
            pltpu.VMEM((SLICE_C,), jnp.float32),
            pltpu.VMEM((SLICE_C,), jnp.int32),
            pltpu.VMEM_SHARED((16 * NPAD,), jnp.float32),
            pltpu.VMEM_SHARED((16 * NPAD,), jnp.int32),
            pltpu.SemaphoreType.DMA,
        ],
    )
    return f(det, src, dst)


def _merge_and_arg(pv_hbm, pa_hbm, v0, v1, a0, a1, argm, off, sem):
    stage = [pltpu.async_copy(pv_hbm.at[pl.ds(off, SLICE_W)], v0, sem),
             pltpu.async_copy(pv_hbm.at[pl.ds(NPAD + off, SLICE_W)], v1, sem),
             pltpu.async_copy(pa_hbm.at[pl.ds(off, SLICE_W)], a0, sem),
             pltpu.async_copy(pa_hbm.at[pl.ds(NPAD + off, SLICE_W)], a1, sem)]
    for d in stage:
        d.wait()

    def mj(j, _):
        sl = pl.ds(j * 16, 16)
        b = _lex_better(v1[sl], a1[sl], v0[sl], a0[sl])
        am = jnp.where(b, a1[sl], a0[sl])
        argm[sl] = jnp.maximum(am, 0)
        return 0
    lax.fori_loop(0, SLICE_W // 16, mj, 0)


def _gather_rows_1d(tab_hbm, idx_ref, out_ref, sem):
    descs = []
    for ch in range(SLICE_W // GCHUNK):
        sl = pl.ds(ch * GCHUNK, GCHUNK)
        descs.append(pltpu.async_copy(tab_hbm.at[idx_ref.at[sl]],
                                      out_ref.at[sl], sem))
    for d in descs:
        d.wait()


def _sc_merge1(pv, pa, dst, det):
    def body(pv_hbm, pa_hbm, dst_hbm, det_hbm, sel_hbm, det2_hbm,
             v0, v1, a0, a1, argm, sel_v, dtab, det2_v, sem):
        cid = lax.axis_index("c")
        sid = lax.axis_index("s")
        wid = cid * 16 + sid
        off = wid * SLICE_W
        _merge_and_arg(pv_hbm, pa_hbm, v0, v1, a0, a1, argm, off, sem)
        _gather_rows_1d(dst_hbm, argm, sel_v, sem)
        pltpu.sync_copy(det_hbm, dtab)

        def gj(j, _):
            sl = pl.ds(j * 16, 16)
            det2_v[sl] = plsc.load_gather(dtab, [sel_v[sl]])
            return 0
        lax.fori_loop(0, SLICE_W // 16, gj, 0)
        pltpu.sync_copy(sel_v, sel_hbm.at[pl.ds(off, SLICE_W)])
        pltpu.sync_copy(det2_v, det2_hbm.at[pl.ds(off, SLICE_W)])

    f = pl.kernel(
        body,
        out_type=(jax.ShapeDtypeStruct((NPAD,), jnp.int32),
                  jax.ShapeDtypeStruct((NPAD,), jnp.float32)),
        mesh=_mesh(),
        compiler_params=pltpu.CompilerParams(needs_layout_passes=False),
        scratch_types=[
            pltpu.VMEM((SLICE_W,), jnp.float32),
            pltpu.VMEM((SLICE_W,), jnp.float32),
            pltpu.VMEM((SLICE_W,), jnp.int32),
            pltpu.VMEM((SLICE_W,), jnp.int32),
            pltpu.VMEM((SLICE_W,), jnp.int32),
            pltpu.VMEM((SLICE_W,), jnp.int32),
            pltpu.VMEM((NPAD,), jnp.float32),
            pltpu.VMEM((SLICE_W,), jnp.float32),
            pltpu.SemaphoreType.DMA,
        ],
    )
    return f(pv, pa, dst, det)


def _sc_final(pv, pa, dst, sel1, tbl, kcol):
    def body(pv_hbm, pa_hbm, dst_hbm, sel1_hbm, tbl_hbm, kcol_hbm,
             rows_hbm, kl_hbm,
             v0, v1, a0, a1, argm, sel2_v, s1tab, ktab, sf_v, kl_v,
             rows_v, sem):
        cid = lax.axis_index("c")
        sid = lax.axis_index("s")
        wid = cid * 16 + sid
        off = wid * SLICE_W
        _merge_and_arg(pv_hbm, pa_hbm, v0, v1, a0, a1, argm, off, sem)
        _gather_rows_1d(dst_hbm, argm, sel2_v, sem)
        pltpu.sync_copy(sel1_hbm, s1tab)
        pltpu.sync_copy(kcol_hbm, ktab)

        def gj(j, _):
            sl = pl.ds(j * 16, 16)
            sf = plsc.load_gather(s1tab, [sel2_v[sl]])
            sf_v[sl] = sf
            kl_v[sl] = plsc.load_gather(ktab, [sf])
            return 0
        lax.fori_loop(0, SLICE_W // 16, gj, 0)

        descs = []
        for ch in range(SLICE_W // GCHUNK):
            sl = pl.ds(ch * GCHUNK, GCHUNK)
            descs.append(pltpu.async_copy(tbl_hbm.at[sf_v.at[sl]],
                                          rows_v.at[sl, :], sem))
        for d in descs:
            d.wait()
        pltpu.sync_copy(rows_v, rows_hbm.at[pl.ds(off, SLICE_W), :])
        pltpu.sync_copy(kl_v, kl_hbm.at[pl.ds(off, SLICE_W)])

    f = pl.kernel(
        body,
        out_type=(jax.ShapeDtypeStruct((NPAD, 16), jnp.float32),
                  jax.ShapeDtypeStruct((NPAD,), jnp.float32)),
        mesh=_mesh(),
        compiler_params=pltpu.CompilerParams(needs_layout_passes=False,
                                             use_tc_tiling_on_sc=False),
        scratch_types=[
            pltpu.VMEM((SLICE_W,), jnp.float32),
            pltpu.VMEM((SLICE_W,), jnp.float32),
            pltpu.VMEM((SLICE_W,), jnp.int32),
            pltpu.VMEM((SLICE_W,), jnp.int32),
            pltpu.VMEM((SLICE_W,), jnp.int32),
            pltpu.VMEM((SLICE_W,), jnp.int32),
            pltpu.VMEM((NPAD,), jnp.int32),
            pltpu.VMEM((NPAD,), jnp.float32),
            pltpu.VMEM((SLICE_W,), jnp.int32),
            pltpu.VMEM((SLICE_W,), jnp.float32),
            pltpu.VMEM((SLICE_W, 16), jnp.float32),
            pltpu.SemaphoreType.DMA,
        ],
    )
    return f(pv, pa, dst, sel1, tbl, kcol)


def kernel(emb, W_ext, b_ext, eps1, eps2, edge_index):
    N = emb.shape[0]
    embT = jnp.pad(emb, ((0, NPAD - N), (0, 0)))
    WT = W_ext.T
    b2 = b_ext[:, None]
    e1T = jnp.pad(eps1[:, :, 0], ((0, NPAD - N), (0, 0))).T
    e2T = jnp.pad(eps2[:, :, 0], ((0, NPAD - N), (0, 0))).T

    detT, predT, zT, kcolT = _dense_tc(embT, WT, b2, e1T, e2T)
    det1 = detT[0]
    z = zT.T[:N]
    tbl = predT.T
    kcol = kcolT[0]

    src_n = edge_index[0]
    dst_n = edge_index[1]
    pv, pa = _sc_partial(det1, src_n, dst_n)
    sel1, det2 = _sc_merge1(pv, pa, dst_n, det1)
    pv2, pa2 = _sc_partial(det2, src_n, dst_n)
    rows, klsel = _sc_final(pv2, pa2, dst_n, sel1, tbl, kcol)

    return jnp.concatenate([rows[:N], z, klsel[:N, None]], axis=1)
